# baseline (speedup 1.0000x reference)
import functools

import jax
import jax.numpy as jnp
from jax import lax
from jax.experimental import pallas as pl
from jax.experimental.pallas import tpu as pltpu
from jax.experimental.pallas import tpu_sc as plsc

N = 10000
D = 128
NCOL = 64
TRASH = 240
NP = N + TRASH
RPT = NP // 16
QROWS = RPT // 4
CHUNK = 128


@functools.cache
def _mesh():
    return plsc.VectorSubcoreMesh(core_axis_name="c", subcore_axis_name="s",
                                  num_cores=2, num_subcores=16)


def _sc_body(streams, part, *refs):
    if part == "a":
        (x_h, srcall_h, dst_h, z_h, z1_h, one_h,
         out1_h, out2_h, out3_h, s_h, deg_h,
         acc, dacc, sidx, didx, r0, r1, r2, r3,
         tbuf, dbuf, dsm, ones_v,
         g0, g1, g2, g3, s0, s1, s2, s3) = refs
    else:
        (out3_h, srcall_h, dst_h, z_h, deg_h,
         xh_h,
         acc, sidx, didx, r0, r1, r2, r3,
         tbuf, dbuf, dsm,
         g0, g1, g2, g3, s0, s1, s2, s3) = refs
        dacc = ones_v = None
    bufs = (r0, r1, r2, r3)
    gsems = (g0, g1, g2, g3)
    ssems = (s0, s1, s2, s3)
    c = lax.axis_index("c")
    s = lax.axis_index("s")
    base = s * RPT
    half = streams // 2

    for q in range(4):
        pltpu.sync_copy(z_h, acc.at[pl.ds(base + q * QROWS, QROWS)])
    if part == "a":
        pltpu.sync_copy(z1_h, dacc.at[pl.ds(base, RPT)])
        pltpu.sync_copy(one_h, ones_v)
    else:
        pltpu.sync_copy(deg_h.at[pl.ds(base, RPT)], dbuf)

        @pl.loop(0, RPT // 16)
        def _(i):
            v = dbuf[pl.ds(i * 16, 16)]
            iv = 1.0 / jnp.maximum(v, 1.0)
            for rr in range(16):
                dsm[i * 16 + rr] = iv[rr]

    def run_layer(in_h, out_h, wide_out, with_deg):
        plsc.subcore_barrier()

        def start_gather(k, b):
            pltpu.async_copy(in_h.at[sidx.at[k]], bufs[b], gsems[b])

        def wait_gather(k, b):
            pltpu.make_async_copy(in_h.at[sidx.at[k]], bufs[b],
                                  gsems[b]).wait()

        def start_scatter(k, b):
            pltpu.async_copy(bufs[b], acc.at[didx.at[k]], ssems[b], add=True)

        def wait_scatter(k, b):
            pltpu.make_async_copy(bufs[b], acc.at[didx.at[k]],
                                  ssems[b]).wait()

        pltpu.sync_copy(srcall_h.at[c, pl.ds(s * streams, streams)], sidx)
        pltpu.sync_copy(dst_h.at[pl.ds(s * streams, streams)], didx)
        start_gather(0, 0)
        start_gather(1, 1)

        @pl.loop(0, streams // 4)
        def _(g):
            for j in range(4):
                k = 4 * g + j
                wait_gather(k, j)
                start_scatter(k, j)
                if with_deg:
                    pltpu.sync_copy(ones_v, dacc.at[didx.at[k]], add=True)
                bn = (j + 2) % 4
                if j < 2:
                    @pl.when(g > 0)
                    def _():
                        wait_scatter(k - 2, bn)
                else:
                    wait_scatter(k - 2, bn)

                @pl.when(k + 2 < streams)
                def _():
                    start_gather(k + 2, bn)

        wait_scatter(streams - 2, 2)
        wait_scatter(streams - 1, 3)

        plsc.subcore_barrier()

        if with_deg:
            pltpu.sync_copy(dacc.at[pl.ds(base, RPT)], dbuf)

            @pl.when(c == 0)
            def _():
                pltpu.sync_copy(dbuf, deg_h.at[pl.ds(base, RPT)])

            @pl.loop(0, RPT // 16)
            def _(i):
                v = dbuf[pl.ds(i * 16, 16)]
                iv = 1.0 / jnp.maximum(v, 1.0)
                for rr in range(16):
                    dsm[i * 16 + rr] = iv[rr]

        for q in range(4):
            rb = base + q * QROWS
            pltpu.sync_copy(acc.at[pl.ds(rb, QROWS)], tbuf)
            pltpu.sync_copy(z_h, acc.at[pl.ds(rb, QROWS)])

            @pl.loop(0, QROWS // 4)
            def _(rq):
                for u in range(4):
                    r = rq * 4 + u
                    d = dsm[q * QROWS + r]
                    for j in range(NCOL // 16):
                        v = tbuf[r, pl.ds(j * 16, 16)]
                        tbuf[r, pl.ds(j * 16, 16)] = jnp.maximum(v * d, 0.0)

            if out_h is not None:
                pltpu.sync_copy(tbuf, out_h.at[pl.ds(c * NP + rb, QROWS)])
            if wide_out is not None:
                pltpu.sync_copy(
                    tbuf,
                    wide_out.at[pl.ds(rb, QROWS), pl.ds(c * NCOL, NCOL)])

    if part == "a":
        run_layer(x_h, out1_h, None, True)
        run_layer(out1_h, out2_h, None, False)
        run_layer(out2_h, out3_h, s_h, False)
    else:
        run_layer(out3_h, None, xh_h, False)


def _make_sc(streams, part):
    f32 = jnp.float32
    if part == "a":
        out_type = ([jax.ShapeDtypeStruct((2 * NP, NCOL), f32)
                     for _ in range(3)]
                    + [jax.ShapeDtypeStruct((NP, D), f32),
                       jax.ShapeDtypeStruct((NP,), f32)])
    else:
        out_type = [jax.ShapeDtypeStruct((NP, D), f32)]
    scratch = [pltpu.VMEM_SHARED((NP, NCOL), f32)]
    if part == "a":
        scratch += [pltpu.VMEM_SHARED((NP,), f32)]
    scratch += [
        pltpu.VMEM((streams, CHUNK), jnp.int32),
        pltpu.VMEM((streams, CHUNK), jnp.int32),
        pltpu.VMEM((CHUNK, NCOL), f32),
        pltpu.VMEM((CHUNK, NCOL), f32),
        pltpu.VMEM((CHUNK, NCOL), f32),
        pltpu.VMEM((CHUNK, NCOL), f32),
        pltpu.VMEM((QROWS, NCOL), f32),
        pltpu.VMEM((RPT,), f32),
        pltpu.SMEM((RPT,), f32),
    ]
    if part == "a":
        scratch += [pltpu.VMEM((CHUNK,), f32)]
    scratch += [pltpu.SemaphoreType.DMA] * 8
    return pl.kernel(
        functools.partial(_sc_body, streams, part),
        out_type=out_type,
        mesh=_mesh(),
        scratch_types=scratch,
        compiler_params=pltpu.CompilerParams(use_tc_tiling_on_sc=False),
    )


def _mm_body(a, b, out):
    out[...] = lax.dot_general(a[...], b[...], (((1,), (1,)), ((), ())),
                               preferred_element_type=jnp.float32)


def _matmul(s_p):
    bm, bn = 2048, 2048
    return pl.pallas_call(
        _mm_body,
        grid=(NP // bm, NP // bn),
        in_specs=[
            pl.BlockSpec((bm, D), lambda i, j: (i, 0)),
            pl.BlockSpec((bn, D), lambda i, j: (j, 0)),
        ],
        out_specs=pl.BlockSpec((bm, bn), lambda i, j: (i, j)),
        out_shape=jax.ShapeDtypeStruct((N, N), jnp.float32),
    )(s_p, s_p)


def kernel(x, edge_index):
    f32 = jnp.float32
    src = edge_index[0].astype(jnp.int32)
    dst = edge_index[1].astype(jnp.int32)
    e = src.shape[0]
    per = 16 * CHUNK * 8
    ep = ((e + per - 1) // per) * per
    padn = ep - e
    pad_ids = jnp.arange(padn, dtype=jnp.int32)
    srcp = jnp.concatenate([src, (pad_ids * 37) % N]).reshape(ep // CHUNK, CHUNK)
    dstp = jnp.concatenate([dst, N + pad_ids % TRASH]).reshape(ep // CHUNK, CHUNK)
    srcall = jnp.stack([srcp, srcp + NP])
    streams = ep // CHUNK // 16

    zrow = jnp.zeros((NP - N, NCOL), f32)
    xcat = jnp.concatenate([x[:, :NCOL], zrow, x[:, NCOL:], zrow])
    z_h = jnp.zeros((QROWS, NCOL), f32)
    z1_h = jnp.zeros((RPT,), f32)
    one_h = jnp.ones((CHUNK,), f32)

    _, _, out3, s_p, deg = _make_sc(streams, "a")(xcat, srcall, dstp,
                                                  z_h, z1_h, one_h)
    xh_p, = _make_sc(streams, "b")(out3, srcall, dstp, z_h, deg)
    a_hat = _matmul(s_p)
    return a_hat, xh_p[:N]


# speedup vs baseline: 1.0742x; 1.0742x over previous
import functools

import jax
import jax.numpy as jnp
from jax import lax
from jax.experimental import pallas as pl
from jax.experimental.pallas import tpu as pltpu
from jax.experimental.pallas import tpu_sc as plsc

N = 10000
D = 128
NCOL = 64
TRASH = 240
NP = N + TRASH
RPT = NP // 16
QROWS = RPT // 4
CHUNK = 128


@functools.cache
def _mesh():
    return plsc.VectorSubcoreMesh(core_axis_name="c", subcore_axis_name="s",
                                  num_cores=2, num_subcores=16)


def _sc_body(streams, part, *refs):
    if part == "a":
        (x_h, srcall_h, dst_h, z_h, z1_h, one_h,
         out1_h, out2_h, out3_h, s_h, deg_h,
         acc, dacc, sidx, didx, r0, r1, r2, r3, r4,
         tbuf, dbuf, dsm, ones_v,
         g0, g1, g2, g3, g4, s0, s1, s2, s3, s4) = refs
    else:
        (out3_h, srcall_h, dst_h, z_h, deg_h,
         xh_h,
         acc, sidx, didx, r0, r1, r2, r3, r4,
         tbuf, dbuf, dsm,
         g0, g1, g2, g3, g4, s0, s1, s2, s3, s4) = refs
        dacc = ones_v = None
    bufs = (r0, r1, r2, r3, r4)
    gsems = (g0, g1, g2, g3, g4)
    ssems = (s0, s1, s2, s3, s4)
    c = lax.axis_index("c")
    s = lax.axis_index("s")
    base = s * RPT
    half = streams // 2

    for q in range(4):
        pltpu.sync_copy(z_h, acc.at[pl.ds(base + q * QROWS, QROWS)])
    if part == "a":
        pltpu.sync_copy(z1_h, dacc.at[pl.ds(base, RPT)])
        pltpu.sync_copy(one_h, ones_v)
    else:
        pltpu.sync_copy(deg_h.at[pl.ds(base, RPT)], dbuf)

        @pl.loop(0, RPT // 16)
        def _(i):
            v = dbuf[pl.ds(i * 16, 16)]
            iv = 1.0 / jnp.maximum(v, 1.0)
            for rr in range(16):
                dsm[i * 16 + rr] = iv[rr]

    def run_layer(in_h, out_h, wide_out, with_deg):
        plsc.subcore_barrier()

        def start_gather(k, b):
            pltpu.async_copy(in_h.at[sidx.at[k]], bufs[b], gsems[b])

        def wait_gather(k, b):
            pltpu.make_async_copy(in_h.at[sidx.at[k]], bufs[b],
                                  gsems[b]).wait()

        def start_scatter(k, b):
            pltpu.async_copy(bufs[b], acc.at[didx.at[k]], ssems[b], add=True)

        def wait_scatter(k, b):
            pltpu.make_async_copy(bufs[b], acc.at[didx.at[k]],
                                  ssems[b]).wait()

        for h in range(2):
            irow = s * streams + h * half
            pltpu.sync_copy(srcall_h.at[c, pl.ds(irow, half)], sidx)
            pltpu.sync_copy(dst_h.at[pl.ds(irow, half)], didx)
            for j in range(3):
                start_gather(j, j)

            @pl.loop(0, half // 5)
            def _(g):
                for j in range(5):
                    k = 5 * g + j
                    wait_gather(k, j)
                    start_scatter(k, j)
                    if with_deg:
                        pltpu.sync_copy(ones_v, dacc.at[didx.at[k]], add=True)
                    bn = (j + 3) % 5
                    if j < 2:
                        @pl.when(g > 0)
                        def _():
                            wait_scatter(k - 2, bn)
                    else:
                        wait_scatter(k - 2, bn)

                    @pl.when(k + 3 < half)
                    def _():
                        start_gather(k + 3, bn)

            wait_scatter(half - 2, (half - 2) % 5)
            wait_scatter(half - 1, (half - 1) % 5)

        plsc.subcore_barrier()

        if with_deg:
            pltpu.sync_copy(dacc.at[pl.ds(base, RPT)], dbuf)

            @pl.when(c == 0)
            def _():
                pltpu.sync_copy(dbuf, deg_h.at[pl.ds(base, RPT)])

            @pl.loop(0, RPT // 16)
            def _(i):
                v = dbuf[pl.ds(i * 16, 16)]
                iv = 1.0 / jnp.maximum(v, 1.0)
                for rr in range(16):
                    dsm[i * 16 + rr] = iv[rr]

        for q in range(4):
            rb = base + q * QROWS
            pltpu.sync_copy(acc.at[pl.ds(rb, QROWS)], tbuf)
            pltpu.sync_copy(z_h, acc.at[pl.ds(rb, QROWS)])

            @pl.loop(0, QROWS // 4)
            def _(rq):
                for u in range(4):
                    r = rq * 4 + u
                    d = dsm[q * QROWS + r]
                    for j in range(NCOL // 16):
                        v = tbuf[r, pl.ds(j * 16, 16)]
                        tbuf[r, pl.ds(j * 16, 16)] = jnp.maximum(v * d, 0.0)

            if out_h is not None:
                pltpu.sync_copy(tbuf, out_h.at[pl.ds(c * NP + rb, QROWS)])
            if wide_out is not None:
                pltpu.sync_copy(
                    tbuf,
                    wide_out.at[pl.ds(rb, QROWS), pl.ds(c * NCOL, NCOL)])

    if part == "a":
        run_layer(x_h, out1_h, None, True)
        run_layer(out1_h, out2_h, None, False)
        run_layer(out2_h, out3_h, s_h, False)
    else:
        run_layer(out3_h, None, xh_h, False)


def _make_sc(streams, part):
    f32 = jnp.float32
    if part == "a":
        out_type = ([jax.ShapeDtypeStruct((2 * NP, NCOL), f32)
                     for _ in range(3)]
                    + [jax.ShapeDtypeStruct((NP, D), f32),
                       jax.ShapeDtypeStruct((NP,), f32)])
    else:
        out_type = [jax.ShapeDtypeStruct((NP, D), f32)]
    scratch = [pltpu.VMEM_SHARED((NP, NCOL), f32)]
    if part == "a":
        scratch += [pltpu.VMEM_SHARED((NP,), f32)]
    scratch += [
        pltpu.VMEM((streams // 2, CHUNK), jnp.int32),
        pltpu.VMEM((streams // 2, CHUNK), jnp.int32),
        pltpu.VMEM((CHUNK, NCOL), f32),
        pltpu.VMEM((CHUNK, NCOL), f32),
        pltpu.VMEM((CHUNK, NCOL), f32),
        pltpu.VMEM((CHUNK, NCOL), f32),
        pltpu.VMEM((CHUNK, NCOL), f32),
        pltpu.VMEM((QROWS, NCOL), f32),
        pltpu.VMEM((RPT,), f32),
        pltpu.SMEM((RPT,), f32),
    ]
    if part == "a":
        scratch += [pltpu.VMEM((CHUNK,), f32)]
    scratch += [pltpu.SemaphoreType.DMA] * 10
    return pl.kernel(
        functools.partial(_sc_body, streams, part),
        out_type=out_type,
        mesh=_mesh(),
        scratch_types=scratch,
        compiler_params=pltpu.CompilerParams(use_tc_tiling_on_sc=False),
    )


def _mm_body(a, b, out):
    out[...] = lax.dot_general(a[...], b[...], (((1,), (1,)), ((), ())),
                               preferred_element_type=jnp.float32)


def _matmul(s_p):
    bm, bn = 2048, 2048
    return pl.pallas_call(
        _mm_body,
        grid=(NP // bm, NP // bn),
        in_specs=[
            pl.BlockSpec((bm, D), lambda i, j: (i, 0)),
            pl.BlockSpec((bn, D), lambda i, j: (j, 0)),
        ],
        out_specs=pl.BlockSpec((bm, bn), lambda i, j: (i, j)),
        out_shape=jax.ShapeDtypeStruct((N, N), jnp.float32),
    )(s_p, s_p)


def kernel(x, edge_index):
    f32 = jnp.float32
    src = edge_index[0].astype(jnp.int32)
    dst = edge_index[1].astype(jnp.int32)
    e = src.shape[0]
    per = 16 * CHUNK * 8
    ep = ((e + per - 1) // per) * per
    padn = ep - e
    pad_ids = jnp.arange(padn, dtype=jnp.int32)
    srcp = jnp.concatenate([src, (pad_ids * 37) % N]).reshape(ep // CHUNK, CHUNK)
    dstp = jnp.concatenate([dst, N + pad_ids % TRASH]).reshape(ep // CHUNK, CHUNK)
    srcall = jnp.stack([srcp, srcp + NP])
    streams = ep // CHUNK // 16

    zrow = jnp.zeros((NP - N, NCOL), f32)
    xcat = jnp.concatenate([x[:, :NCOL], zrow, x[:, NCOL:], zrow])
    z_h = jnp.zeros((QROWS, NCOL), f32)
    z1_h = jnp.zeros((RPT,), f32)
    one_h = jnp.ones((CHUNK,), f32)

    _, _, out3, s_p, deg = _make_sc(streams, "a")(xcat, srcall, dstp,
                                                  z_h, z1_h, one_h)
    xh_p, = _make_sc(streams, "b")(out3, srcall, dstp, z_h, deg)
    a_hat = _matmul(s_p)
    return a_hat, xh_p[:N]
